# baseline (speedup 1.0000x reference)
import functools

import jax
import jax.numpy as jnp
from jax import lax
from jax.experimental import pallas as pl
from jax.experimental.pallas import tpu as pltpu
from jax.experimental.pallas import tpu_sc as plsc

B, S, D = 4, 4096, 2048
K = S // 2
MININT = -(2**31)

NB = 2
ROWS_PER_STEP = 1024
BATCH_ROWS = S // ROWS_PER_STEP
HGRID = NB * S // ROWS_PER_STEP

ROWS_PER_TILE = K // 16
GATHER_CHUNK = 16
N_CHUNKS = ROWS_PER_TILE // GATHER_CHUNK


def _sortable_i32(f32_arr):
    bits = lax.bitcast_convert_type(f32_arr, jnp.int32)
    return jnp.where(bits >= 0, bits, bits ^ jnp.int32(0x7FFFFFFF))


def _scores_thr_body(x_ref, w_ref, b_ref, s_ref, k_ref, thr_ref, acc_ref):
    j = pl.program_id(0)
    x16 = x_ref[...].astype(jnp.bfloat16)
    w16 = w_ref[...].astype(jnp.bfloat16)
    m = lax.dot_general(w16, x16, (((1,), (1,)), ((), ())),
                        preferred_element_type=jnp.float32)
    sb = m + b_ref[0]
    s_ref[...] = sb[:, None, :]
    k_ref[...] = _sortable_i32(sb)[:, None, :]
    acc_ref[pl.ds(j // BATCH_ROWS, 1),
            pl.ds((j % BATCH_ROWS) * ROWS_PER_STEP, ROWS_PER_STEP)] = sb

    @pl.when(j == HGRID - 1)
    def _():
        skey = _sortable_i32(acc_ref[...])

        def body(i, pat):
            bit = lax.shift_left(jnp.int32(1), jnp.int32(31) - i)
            cand = pat | bit
            thr_s = cand ^ jnp.int32(MININT)
            cnt = jnp.sum((skey >= thr_s).astype(jnp.int32), axis=1,
                          keepdims=True)
            return jnp.where(cnt >= K, cand, pat)

        pat = lax.fori_loop(0, 32, body, jnp.zeros((NB, 1), jnp.int32))
        thr_s = pat ^ jnp.int32(MININT)
        cnt_gt = jnp.sum((skey > thr_s).astype(jnp.int32), axis=1,
                         keepdims=True)
        ne = K - cnt_gt
        thr_ref[...] = jnp.concatenate(
            [jnp.broadcast_to(thr_s, (NB, 128)),
             jnp.broadcast_to(ne, (NB, 128))], axis=0)


def _scores_and_thresholds(x2, W, b, half):
    row0 = half * HGRID
    scores3, skeys3, thr = pl.pallas_call(
        _scores_thr_body,
        grid=(HGRID,),
        in_specs=[
            pl.BlockSpec((ROWS_PER_STEP, D), lambda j: (j + row0, 0)),
            pl.BlockSpec((1, D), lambda j: (0, 0)),
            pl.BlockSpec(memory_space=pltpu.SMEM),
        ],
        out_specs=[
            pl.BlockSpec((1, 1, ROWS_PER_STEP), lambda j: (j, 0, 0)),
            pl.BlockSpec((1, 1, ROWS_PER_STEP), lambda j: (j, 0, 0)),
            pl.BlockSpec((2 * NB, 128), lambda j: (0, 0)),
        ],
        out_shape=[
            jax.ShapeDtypeStruct((HGRID, 1, ROWS_PER_STEP), jnp.float32),
            jax.ShapeDtypeStruct((HGRID, 1, ROWS_PER_STEP), jnp.int32),
            jax.ShapeDtypeStruct((2 * NB, 128), jnp.int32),
        ],
        scratch_shapes=[pltpu.VMEM((NB, S), jnp.float32)],
    )(x2, W, b)
    return scores3.reshape(NB, S), skeys3.reshape(NB, S), thr


def _sc_select_gather(x2, skeys, thr, half):
    mesh = plsc.VectorSubcoreMesh(core_axis_name="c", subcore_axis_name="s")

    @functools.partial(
        pl.kernel,
        out_type=jax.ShapeDtypeStruct((NB * K, D), jnp.float32),
        mesh=mesh,
        compiler_params=pltpu.CompilerParams(needs_layout_passes=False),
        scratch_types=[
            pltpu.VMEM((S,), jnp.int32),
            pltpu.VMEM((128,), jnp.int32),
            pltpu.VMEM((128,), jnp.int32),
            pltpu.VMEM((K,), jnp.int32),
            pltpu.VMEM_SHARED((K,), jnp.int32),
            pltpu.VMEM((ROWS_PER_TILE,), jnp.int32),
            pltpu.VMEM((GATHER_CHUNK, D), jnp.float32),
            pltpu.VMEM((GATHER_CHUNK, D), jnp.float32),
            pltpu.VMEM((GATHER_CHUNK, D), jnp.float32),
            pltpu.SemaphoreType.DMA,
            pltpu.SemaphoreType.DMA,
            pltpu.SemaphoreType.DMA,
            pltpu.SemaphoreType.DMA,
            pltpu.SemaphoreType.DMA,
            pltpu.SemaphoreType.DMA,
        ],
    )
    def sc_kernel(x_hbm, sc_hbm, thr_hbm, out_hbm, sc_v, thr_v, ne_v, idx_v,
                  idx_sh, idxc_v, buf0, buf1, buf2, gs0, gs1, gs2,
                  ss0, ss1, ss2):
        c = lax.axis_index("c")
        s = lax.axis_index("s")

        @pl.when(s == 0)
        def _build_indices():
            pltpu.sync_copy(sc_hbm.at[c], sc_v)
            pltpu.sync_copy(thr_hbm.at[c], thr_v)
            pltpu.sync_copy(thr_hbm.at[NB + c], ne_v)
            t_vec = thr_v[pl.ds(0, 16)]
            ne_vec = ne_v[pl.ds(0, 16)]
            row0 = (2 * half + c) * S

            def body(i, carry):
                off_vec, eqt_vec = carry
                skey = sc_v[pl.ds(i * 16, 16)]
                gt = skey > t_vec
                eq = skey == t_vec
                eq_rank = eqt_vec + plsc.cumsum(eq.astype(jnp.int32))
                inc = gt | (eq & (eq_rank <= ne_vec))
                pos = off_vec + plsc.cumsum(inc.astype(jnp.int32)) - 1
                gids = lax.iota(jnp.int32, 16) + (row0 + i * 16)
                plsc.store_scatter(idx_v, [pos], gids, mask=inc)
                off_vec = off_vec + plsc.all_reduce_population_count(inc)
                eqt_vec = eqt_vec + plsc.all_reduce_population_count(eq)
                return (off_vec, eqt_vec)

            zero = jnp.zeros((16,), jnp.int32)
            lax.fori_loop(0, S // 16, body, (zero, zero))
            pltpu.sync_copy(idx_v, idx_sh)

        plsc.subcore_barrier()

        ro = s * ROWS_PER_TILE
        pltpu.sync_copy(idx_sh.at[pl.ds(ro, ROWS_PER_TILE)], idxc_v)
        out_base = c * K + ro

        NBUF = 3
        bufs = (buf0, buf1, buf2)
        gsems = (gs0, gs1, gs2)
        ssems = (ss0, ss1, ss2)
        gath = [None] * NBUF
        scat = [None] * NBUF
        for g in range(NBUF - 1):
            iv = idxc_v[pl.ds(g * GATHER_CHUNK, GATHER_CHUNK)]
            gath[g] = pltpu.async_copy(x_hbm.at[iv], bufs[g], gsems[g])
        for g in range(N_CHUNKS):
            p = g % NBUF
            pre = g + NBUF - 1
            if pre < N_CHUNKS:
                q = pre % NBUF
                if scat[q] is not None:
                    scat[q].wait()
                iv = idxc_v[pl.ds(pre * GATHER_CHUNK, GATHER_CHUNK)]
                gath[q] = pltpu.async_copy(x_hbm.at[iv], bufs[q], gsems[q])
            gath[p].wait()
            scat[p] = pltpu.async_copy(
                bufs[p],
                out_hbm.at[pl.ds(out_base + g * GATHER_CHUNK, GATHER_CHUNK)],
                ssems[p])
        for p in range(NBUF):
            if scat[p] is not None:
                scat[p].wait()

    return sc_kernel(x2, skeys, thr)


def kernel(x, W, b):
    x2 = x.reshape(B * S, D)
    outs = []
    scs = []
    for half in range(2):
        scores_h, skeys_h, thr_h = _scores_and_thresholds(x2, W, b, half)
        outs.append(_sc_select_gather(x2, skeys_h, thr_h, half))
        scs.append(scores_h)
    x_sliced = jnp.concatenate(outs, axis=0).reshape(B, K, D)
    scores = jnp.concatenate(scs, axis=0)
    return (x_sliced, scores)


# speedup vs baseline: 1.3913x; 1.3913x over previous
import functools

import jax
import jax.numpy as jnp
from jax import lax
from jax.experimental import pallas as pl
from jax.experimental.pallas import tpu as pltpu
from jax.experimental.pallas import tpu_sc as plsc

B, S, D = 4, 4096, 2048
K = S // 2
S_TILE = 256
GRID = S // S_TILE
MININT = -(2**31)

ROWS_PER_TILE = 2 * K // 16
GATHER_CHUNK = 16
N_CHUNKS = ROWS_PER_TILE // GATHER_CHUNK


def _sortable_i32(f32_arr):
    bits = lax.bitcast_convert_type(f32_arr, jnp.int32)
    return jnp.where(bits >= 0, bits, bits ^ jnp.int32(0x7FFFFFFF))


ROWS_PER_STEP = B * S // GRID
BATCH_ROWS = S // ROWS_PER_STEP


def _scores_thr_body(x_ref, w_ref, b_ref, s_ref, k_ref, thr_ref, acc_ref):
    j = pl.program_id(0)
    x16 = x_ref[...].astype(jnp.bfloat16)
    w16 = w_ref[...].astype(jnp.bfloat16)
    m = lax.dot_general(w16, x16, (((1,), (1,)), ((), ())),
                        preferred_element_type=jnp.float32)
    sb = m + b_ref[0]
    s_ref[...] = sb[:, None, :]
    k_ref[...] = _sortable_i32(sb)[:, None, :]
    acc_ref[pl.ds(j // BATCH_ROWS, 1),
            pl.ds((j % BATCH_ROWS) * ROWS_PER_STEP, ROWS_PER_STEP)] = sb

    @pl.when(j == GRID - 1)
    def _():
        skey = _sortable_i32(acc_ref[...])

        def body(i, pat):
            bit = lax.shift_left(jnp.int32(1), jnp.int32(31) - i)
            cand = pat | bit
            thr_s = cand ^ jnp.int32(MININT)
            cnt = jnp.sum((skey >= thr_s).astype(jnp.int32), axis=1,
                          keepdims=True)
            return jnp.where(cnt >= K, cand, pat)

        pat = lax.fori_loop(0, 32, body, jnp.zeros((B, 1), jnp.int32))
        thr_s = pat ^ jnp.int32(MININT)
        cnt_gt = jnp.sum((skey > thr_s).astype(jnp.int32), axis=1,
                         keepdims=True)
        ne = K - cnt_gt
        thr_ref[...] = jnp.concatenate(
            [jnp.broadcast_to(thr_s, (B, 128)),
             jnp.broadcast_to(ne, (B, 128))], axis=0)


def _scores_and_thresholds(x2, W, b):
    scores3, skeys3, thr = pl.pallas_call(
        _scores_thr_body,
        grid=(GRID,),
        in_specs=[
            pl.BlockSpec((ROWS_PER_STEP, D), lambda j: (j, 0)),
            pl.BlockSpec((1, D), lambda j: (0, 0)),
            pl.BlockSpec(memory_space=pltpu.SMEM),
        ],
        out_specs=[
            pl.BlockSpec((1, 1, ROWS_PER_STEP), lambda j: (j, 0, 0)),
            pl.BlockSpec((1, 1, ROWS_PER_STEP), lambda j: (j, 0, 0)),
            pl.BlockSpec((2 * B, 128), lambda j: (0, 0)),
        ],
        out_shape=[
            jax.ShapeDtypeStruct((GRID, 1, ROWS_PER_STEP), jnp.float32),
            jax.ShapeDtypeStruct((GRID, 1, ROWS_PER_STEP), jnp.int32),
            jax.ShapeDtypeStruct((2 * B, 128), jnp.int32),
        ],
        scratch_shapes=[pltpu.VMEM((B, S), jnp.float32)],
    )(x2, W, b)
    return scores3.reshape(B, S), skeys3.reshape(B, S), thr


def _sc_select_gather(x2, skeys, thr):
    mesh = plsc.VectorSubcoreMesh(core_axis_name="c", subcore_axis_name="s")

    @functools.partial(
        pl.kernel,
        out_type=jax.ShapeDtypeStruct((B * K, D), jnp.float32),
        mesh=mesh,
        compiler_params=pltpu.CompilerParams(needs_layout_passes=False),
        scratch_types=[
            pltpu.VMEM((S,), jnp.int32),
            pltpu.VMEM((128,), jnp.int32),
            pltpu.VMEM((128,), jnp.int32),
            pltpu.VMEM((K,), jnp.int32),
            pltpu.VMEM_SHARED((2 * K,), jnp.int32),
            pltpu.VMEM((ROWS_PER_TILE,), jnp.int32),
            pltpu.VMEM((GATHER_CHUNK, D), jnp.float32),
            pltpu.VMEM((GATHER_CHUNK, D), jnp.float32),
            pltpu.VMEM((GATHER_CHUNK, D), jnp.float32),
            pltpu.SemaphoreType.DMA,
            pltpu.SemaphoreType.DMA,
            pltpu.SemaphoreType.DMA,
            pltpu.SemaphoreType.DMA,
            pltpu.SemaphoreType.DMA,
            pltpu.SemaphoreType.DMA,
        ],
    )
    def sc_kernel(x_hbm, sc_hbm, thr_hbm, out_hbm, sc_v, thr_v, ne_v, idx_v,
                  idx_sh, idxc_v, buf0, buf1, buf2, gs0, gs1, gs2,
                  ss0, ss1, ss2):
        c = lax.axis_index("c")
        s = lax.axis_index("s")

        @pl.when(s < 2)
        def _build_indices():
            b = c * 2 + s
            pltpu.sync_copy(sc_hbm.at[b], sc_v)
            pltpu.sync_copy(thr_hbm.at[b], thr_v)
            pltpu.sync_copy(thr_hbm.at[B + b], ne_v)
            t_vec = thr_v[pl.ds(0, 16)]
            ne_vec = ne_v[pl.ds(0, 16)]
            row0 = b * S

            def body(i, carry):
                off_vec, eqt_vec = carry
                skey = sc_v[pl.ds(i * 16, 16)]
                gt = skey > t_vec
                eq = skey == t_vec
                eq_rank = eqt_vec + plsc.cumsum(eq.astype(jnp.int32))
                inc = gt | (eq & (eq_rank <= ne_vec))
                pos = off_vec + plsc.cumsum(inc.astype(jnp.int32)) - 1
                gids = lax.iota(jnp.int32, 16) + (row0 + i * 16)
                plsc.store_scatter(idx_v, [pos], gids, mask=inc)
                off_vec = off_vec + plsc.all_reduce_population_count(inc)
                eqt_vec = eqt_vec + plsc.all_reduce_population_count(eq)
                return (off_vec, eqt_vec)

            zero = jnp.zeros((16,), jnp.int32)
            lax.fori_loop(0, S // 16, body, (zero, zero))
            pltpu.sync_copy(idx_v, idx_sh.at[pl.ds(s * K, K)])

        plsc.subcore_barrier()

        bl = s // 8
        ro = (s % 8) * ROWS_PER_TILE
        b = c * 2 + bl
        pltpu.sync_copy(idx_sh.at[pl.ds(bl * K + ro, ROWS_PER_TILE)], idxc_v)
        out_base = b * K + ro

        NBUF = 3
        bufs = (buf0, buf1, buf2)
        gsems = (gs0, gs1, gs2)
        ssems = (ss0, ss1, ss2)
        gath = [None] * NBUF
        scat = [None] * NBUF
        for g in range(NBUF - 1):
            iv = idxc_v[pl.ds(g * GATHER_CHUNK, GATHER_CHUNK)]
            gath[g] = pltpu.async_copy(x_hbm.at[iv], bufs[g], gsems[g])
        for g in range(N_CHUNKS):
            p = g % NBUF
            pre = g + NBUF - 1
            if pre < N_CHUNKS:
                q = pre % NBUF
                if scat[q] is not None:
                    scat[q].wait()
                iv = idxc_v[pl.ds(pre * GATHER_CHUNK, GATHER_CHUNK)]
                gath[q] = pltpu.async_copy(x_hbm.at[iv], bufs[q], gsems[q])
            gath[p].wait()
            scat[p] = pltpu.async_copy(
                bufs[p],
                out_hbm.at[pl.ds(out_base + g * GATHER_CHUNK, GATHER_CHUNK)],
                ssems[p])
        for p in range(NBUF):
            if scat[p] is not None:
                scat[p].wait()

    return sc_kernel(x2, skeys, thr)


def kernel(x, W, b):
    x2 = x.reshape(B * S, D)
    scores, skeys, thr = _scores_and_thresholds(x2, W, b)
    out2 = _sc_select_gather(x2, skeys, thr)
    return (out2.reshape(B, K, D), scores)
